# baseline (speedup 1.0000x reference)
import functools

import jax
import jax.numpy as jnp
from jax.experimental import pallas as pl
from jax.experimental.pallas import tpu as pltpu


def _pick_score(scores, expert, num_experts):
    t = scores.shape[0]
    lane = jax.lax.broadcasted_iota(jnp.int32, (t, num_experts), 1)
    return jnp.sum(jnp.where(lane == expert, scores, 0.0), axis=1,
                   keepdims=True)


def _moe_body(x_ref, Wr_ref, br_ref, W1_ref, b1_ref, W2_ref, b2_ref,
              out_ref, scores_ref, xbf_ref, h_ref, *, num_experts, nf, sub):
    s = pl.program_id(0)
    n_chunks = num_experts * nf
    par = s % 2

    @pl.when(s == 0)
    def _init():
        logits = jnp.dot(x_ref[...], Wr_ref[...],
                         preferred_element_type=jnp.float32) + br_ref[...]
        scores_ref[...] = jax.nn.softmax(logits, axis=-1)
        xbf_ref[...] = x_ref[...].astype(jnp.bfloat16)
        out_ref[...] = jnp.zeros_like(out_ref)

    @pl.when(s > 0)
    def _second_matmul():
        sp = s - 1
        e_prev = sp // nf
        f_prev = sp % nf
        w_prev = _pick_score(scores_ref[...], e_prev, num_experts)
        part = jnp.dot(h_ref[1 - par], W2_ref[0].astype(jnp.bfloat16),
                       preferred_element_type=jnp.float32)
        upd = part * w_prev

        @pl.when(f_prev == 0)
        def _bias2():
            out_ref[...] += b2_ref[0] * w_prev

        out_ref[...] += upd

    @pl.when(s < n_chunks)
    def _first_matmul():
        xb = xbf_ref[...]
        fb = W1_ref.shape[2]
        cs = fb // sub
        for i in range(sub):
            sl = slice(i * cs, (i + 1) * cs)
            h = jnp.dot(xb, W1_ref[0, :, sl].astype(jnp.bfloat16),
                        preferred_element_type=jnp.float32)
            h = h + b1_ref[0, :, sl]
            g = jax.lax.erf(h * 0.7071067811865476)
            h_ref[par, :, sl] = (h * (0.5 * g + 0.5)).astype(jnp.bfloat16)


@jax.jit
def kernel(x, Wr, br, W1, b1, W2, b2):
    t, d_model = x.shape
    num_experts, _, d_ff = W1.shape
    f_block = 1536
    nf = d_ff // f_block
    n_chunks = num_experts * nf

    def w1_idx(s):
        m = jnp.minimum(s, n_chunks - 1)
        return (m // nf, 0, m % nf)

    def w2_idx(s):
        m = jnp.maximum(s - 1, 0)
        return (m // nf, m % nf, 0)

    def b2_idx(s):
        m = jnp.maximum(s - 1, 0)
        return (m // nf, 0, 0)

    body = functools.partial(_moe_body, num_experts=num_experts, nf=nf,
                             sub=3)
    out = pl.pallas_call(
        body,
        grid=(n_chunks + 1,),
        in_specs=[
            pl.BlockSpec((t, d_model), lambda s: (0, 0)),
            pl.BlockSpec((d_model, num_experts), lambda s: (0, 0)),
            pl.BlockSpec((1, num_experts), lambda s: (0, 0)),
            pl.BlockSpec((1, d_model, f_block), w1_idx),
            pl.BlockSpec((1, 1, f_block), w1_idx),
            pl.BlockSpec((1, f_block, d_model), w2_idx),
            pl.BlockSpec((1, 1, d_model), b2_idx),
        ],
        out_specs=pl.BlockSpec((t, d_model), lambda s: (0, 0)),
        out_shape=jax.ShapeDtypeStruct((t, d_model), jnp.float32),
        scratch_shapes=[
            pltpu.VMEM((t, num_experts), jnp.float32),
            pltpu.VMEM((t, d_model), jnp.bfloat16),
            pltpu.VMEM((2, t, f_block), jnp.bfloat16),
        ],
        compiler_params=pltpu.CompilerParams(
            dimension_semantics=("arbitrary",),
            vmem_limit_bytes=64 * 1024 * 1024,
        ),
    )(x, Wr, br.reshape(1, num_experts), W1,
      b1.reshape(num_experts, 1, d_ff), W2,
      b2.reshape(num_experts, 1, d_model))
    return out


# speedup vs baseline: 1.1053x; 1.1053x over previous
import functools

import jax
import jax.numpy as jnp
from jax.experimental import pallas as pl
from jax.experimental.pallas import tpu as pltpu


def _moe_body(x_ref, Wr_ref, br_ref, W1_ref, b1_ref, W2_ref, b2_ref,
              out_ref, scores_ref, xbf_ref, w_ref, h_ref, *, num_experts,
              sub):
    e = pl.program_id(0)
    f = pl.program_id(1)

    @pl.when(jnp.logical_and(e == 0, f == 0))
    def _init():
        logits = jnp.dot(x_ref[...], Wr_ref[...],
                         preferred_element_type=jnp.float32) + br_ref[...]
        scores_ref[...] = jax.nn.softmax(logits, axis=-1)
        xbf_ref[...] = x_ref[...].astype(jnp.bfloat16)
        out_ref[...] = jnp.zeros_like(out_ref)

    t = x_ref.shape[0]

    @pl.when(f == 0)
    def _per_expert():
        lane = jax.lax.broadcasted_iota(jnp.int32, (t, num_experts), 1)
        w0 = jnp.sum(jnp.where(lane == e, scores_ref[...], 0.0), axis=1,
                     keepdims=True)
        w_ref[...] = w0
        out_ref[...] += b2_ref[0] * w0

    w = w_ref[...]

    xb = xbf_ref[...]
    fb = W1_ref.shape[2]
    cs = fb // sub
    for i in range(sub):
        sl = slice(i * cs, (i + 1) * cs)
        h = jnp.dot(xb, W1_ref[0, :, sl].astype(jnp.bfloat16),
                    preferred_element_type=jnp.float32)
        h = h + b1_ref[0, :, sl]
        g = jax.lax.erf(h * 0.7071067811865476)
        h_ref[:, sl] = (h * (0.5 * g + 0.5)).astype(jnp.bfloat16)
    part = jnp.dot(h_ref[...], W2_ref[0].astype(jnp.bfloat16),
                   preferred_element_type=jnp.float32)
    out_ref[...] += part * w


@jax.jit
def kernel(x, Wr, br, W1, b1, W2, b2):
    t, d_model = x.shape
    num_experts, _, d_ff = W1.shape
    f_block = 1536
    nf = d_ff // f_block

    body = functools.partial(_moe_body, num_experts=num_experts, sub=3)
    out = pl.pallas_call(
        body,
        grid=(num_experts, nf),
        in_specs=[
            pl.BlockSpec((t, d_model), lambda e, f: (0, 0)),
            pl.BlockSpec((d_model, num_experts), lambda e, f: (0, 0)),
            pl.BlockSpec((1, num_experts), lambda e, f: (0, 0)),
            pl.BlockSpec((1, d_model, f_block), lambda e, f: (e, 0, f)),
            pl.BlockSpec((1, 1, f_block), lambda e, f: (e, 0, f)),
            pl.BlockSpec((1, f_block, d_model), lambda e, f: (e, f, 0)),
            pl.BlockSpec((1, 1, d_model), lambda e, f: (e, 0, 0)),
        ],
        out_specs=pl.BlockSpec((t, d_model), lambda e, f: (0, 0)),
        out_shape=jax.ShapeDtypeStruct((t, d_model), jnp.float32),
        scratch_shapes=[
            pltpu.VMEM((t, num_experts), jnp.float32),
            pltpu.VMEM((t, d_model), jnp.bfloat16),
            pltpu.VMEM((t, 1), jnp.float32),
            pltpu.VMEM((t, f_block), jnp.bfloat16),
        ],
        compiler_params=pltpu.CompilerParams(
            dimension_semantics=("arbitrary", "arbitrary"),
            vmem_limit_bytes=64 * 1024 * 1024,
        ),
    )(x, Wr, br.reshape(1, num_experts), W1,
      b1.reshape(num_experts, 1, d_ff), W2,
      b2.reshape(num_experts, 1, d_model))
    return out
